# baseline (speedup 1.0000x reference)
import jax
import jax.numpy as jnp
from jax.experimental import pallas as pl
from jax.experimental.pallas import tpu as pltpu

_BC = 4


def _body(x_ref, a1_ref, a2_ref, w_ref, b_ref, o_ref):
    f32 = jnp.float32
    bf16 = jnp.bfloat16
    xb = x_ref[:].astype(bf16)
    a1 = a1_ref[:]
    a2 = a2_ref[:]
    w = w_ref[:]
    bias = b_ref[:]

    u1b = jnp.dot(xb, a1, preferred_element_type=f32).astype(bf16)
    v1b = jnp.dot(xb, a2, preferred_element_type=f32).astype(bf16)
    u2b = jnp.dot(u1b, a1, preferred_element_type=f32).astype(bf16)
    v2b = jnp.dot(v1b, a2, preferred_element_type=f32).astype(bf16)

    outs = []
    for j in range(_BC):
        sl = slice(j * 64, (j + 1) * 64)
        h = jnp.concatenate([xb[sl], u1b[sl], u2b[sl], v1b[sl], v2b[sl]],
                            axis=0)
        outs.append(jnp.dot(w, h, preferred_element_type=f32) + bias)
    o_ref[:] = jnp.concatenate(outs, axis=0)


def _run_block(xf, a1, a2, wb, b2d):
    rows_total, N = xf.shape
    C = b2d.shape[0]
    rows = _BC * C
    grid = (rows_total // rows,)
    return pl.pallas_call(
        _body,
        grid=grid,
        in_specs=[
            pl.BlockSpec((rows, N), lambda i: (i, 0)),
            pl.BlockSpec((N, N), lambda i: (0, 0)),
            pl.BlockSpec((N, N), lambda i: (0, 0)),
            pl.BlockSpec((C, 5 * C), lambda i: (0, 0)),
            pl.BlockSpec((C, N), lambda i: (0, 0)),
        ],
        out_specs=pl.BlockSpec((rows, N), lambda i: (i, 0)),
        out_shape=jax.ShapeDtypeStruct((rows_total, N), jnp.float32),
        compiler_params=pltpu.CompilerParams(
            dimension_semantics=("parallel",),
        ),
    )(xf, a1, a2, wb, b2d)


def kernel(x, adjs, W, b):
    B, C, N = x.shape
    xf = x.reshape(B * C, N)
    a1 = adjs[0].astype(jnp.bfloat16)
    a2 = adjs[1].astype(jnp.bfloat16)
    wb = W.astype(jnp.bfloat16)
    b2d = jnp.broadcast_to(b[:, None], (C, N)).astype(jnp.float32)

    out = _run_block(xf, a1, a2, wb, b2d)
    return out.reshape(B, C, N)


# speedup vs baseline: 1.0441x; 1.0441x over previous
import jax
import jax.numpy as jnp
from jax.experimental import pallas as pl
from jax.experimental.pallas import tpu as pltpu

_BC = 8


def _body(x_ref, a1_ref, a2_ref, w_ref, b_ref, o_ref):
    f32 = jnp.float32
    bf16 = jnp.bfloat16
    xb = x_ref[:].astype(bf16)
    a1 = a1_ref[0]
    a2 = a2_ref[0]
    w = w_ref[:]
    bias = b_ref[:]

    u1b = jnp.dot(xb, a1, preferred_element_type=f32).astype(bf16)
    v1b = jnp.dot(xb, a2, preferred_element_type=f32).astype(bf16)
    u2b = jnp.dot(u1b, a1, preferred_element_type=f32).astype(bf16)
    v2b = jnp.dot(v1b, a2, preferred_element_type=f32).astype(bf16)

    outs = []
    for j in range(_BC):
        sl = slice(j * 64, (j + 1) * 64)
        h = jnp.concatenate([xb[sl], u1b[sl], u2b[sl], v1b[sl], v2b[sl]],
                            axis=0)
        outs.append(jnp.dot(w, h, preferred_element_type=f32) + bias)
    o_ref[:] = jnp.concatenate(outs, axis=0)


def kernel(x, adjs, W, b):
    B, C, N = x.shape
    xf = x.reshape(B * C, N)
    adjs_b = adjs.astype(jnp.bfloat16)
    wb = W.astype(jnp.bfloat16)
    b2d = jnp.broadcast_to(b[:, None], (C, N)).astype(jnp.float32)

    rows = _BC * C
    grid = (B // _BC,)
    out = pl.pallas_call(
        _body,
        grid=grid,
        in_specs=[
            pl.BlockSpec((rows, N), lambda i: (i, 0)),
            pl.BlockSpec((1, N, N), lambda i: (0, 0, 0)),
            pl.BlockSpec((1, N, N), lambda i: (1, 0, 0)),
            pl.BlockSpec((C, 5 * C), lambda i: (0, 0)),
            pl.BlockSpec((C, N), lambda i: (0, 0)),
        ],
        out_specs=pl.BlockSpec((rows, N), lambda i: (i, 0)),
        out_shape=jax.ShapeDtypeStruct((B * C, N), jnp.float32),
        compiler_params=pltpu.CompilerParams(
            dimension_semantics=("arbitrary",),
        ),
    )(xf, adjs_b, adjs_b, wb, b2d)
    return out.reshape(B, C, N)


# speedup vs baseline: 1.0606x; 1.0159x over previous
import jax
import jax.numpy as jnp
from jax.experimental import pallas as pl
from jax.experimental.pallas import tpu as pltpu

_BC = 8


def _body(x_ref, a1_ref, a2_ref, w_ref, b_ref, o_ref):
    f32 = jnp.float32
    bf16 = jnp.bfloat16
    xb = x_ref[:].astype(bf16)
    a1 = a1_ref[0]
    a2 = a2_ref[0]
    w = w_ref[:]
    bias = b_ref[:]

    u1b = jnp.dot(xb, a1, preferred_element_type=f32).astype(bf16)
    v1b = jnp.dot(xb, a2, preferred_element_type=f32).astype(bf16)

    w1 = w[:, :192]
    w2 = w[:, 192:]
    parts = []
    for j in range(_BC):
        sl = slice(j * 64, (j + 1) * 64)
        h1 = jnp.concatenate([xb[sl], u1b[sl], v1b[sl]], axis=0)
        parts.append(jnp.dot(w1, h1, preferred_element_type=f32) + bias)

    u2b = jnp.dot(u1b, a1, preferred_element_type=f32).astype(bf16)
    v2b = jnp.dot(v1b, a2, preferred_element_type=f32).astype(bf16)

    outs = []
    for j in range(_BC):
        sl = slice(j * 64, (j + 1) * 64)
        h2 = jnp.concatenate([u2b[sl], v2b[sl]], axis=0)
        outs.append(parts[j] + jnp.dot(w2, h2, preferred_element_type=f32))
    o_ref[:] = jnp.concatenate(outs, axis=0)


def kernel(x, adjs, W, b):
    B, C, N = x.shape
    xf = x.reshape(B * C, N)
    adjs_b = adjs.astype(jnp.bfloat16)
    wb = jnp.concatenate(
        [W[:, 0:128], W[:, 192:256], W[:, 128:192], W[:, 256:320]], axis=1
    ).astype(jnp.bfloat16)
    b2d = jnp.broadcast_to(b[:, None], (C, N)).astype(jnp.float32)

    rows = _BC * C
    grid = (B // _BC,)
    out = pl.pallas_call(
        _body,
        grid=grid,
        in_specs=[
            pl.BlockSpec((rows, N), lambda i: (i, 0)),
            pl.BlockSpec((1, N, N), lambda i: (0, 0, 0)),
            pl.BlockSpec((1, N, N), lambda i: (1, 0, 0)),
            pl.BlockSpec((C, 5 * C), lambda i: (0, 0)),
            pl.BlockSpec((C, N), lambda i: (0, 0)),
        ],
        out_specs=pl.BlockSpec((rows, N), lambda i: (i, 0)),
        out_shape=jax.ShapeDtypeStruct((B * C, N), jnp.float32),
        compiler_params=pltpu.CompilerParams(
            dimension_semantics=("arbitrary",),
        ),
    )(xf, adjs_b, adjs_b, wb, b2d)
    return out.reshape(B, C, N)
